# Initial kernel scaffold; baseline (speedup 1.0000x reference)
import jax
import jax.numpy as jnp
from jax.experimental import pallas as pl


def kernel(x, mask, codebooks):
    raise NotImplementedError("write your pallas kernel here")


# speedup vs baseline: 1.2236x; 1.2236x over previous
import functools

import jax
import jax.numpy as jnp
from jax import lax
from jax.experimental import pallas as pl
from jax.experimental.pallas import tpu as pltpu
from jax.experimental.pallas import tpu_sc as plsc

B, T = 16, 576
N = B * T
K = 8192
D = 64
BN = 512
NBLK = N // BN
BK = 512
NCHUNK = K // BK
LOSS_W = 1.25 / D

SC_CORES = 2
SC_SUBCORES = 16
NW = SC_CORES * SC_SUBCORES
BPW = N // NW


def _argmin_body(xm2_ref, cbt_ref, mask_ref, ind_ref, indg_ref, loss_ref, acc_ref):
    i = pl.program_id(0)
    xm2 = xm2_ref[...]
    xp = jnp.sum(xm2 * xm2, axis=1, keepdims=True) * 0.25

    best_val = jnp.full((BN, 1), jnp.inf, dtype=jnp.float32)
    best_idx = jnp.zeros((BN, 1), dtype=jnp.int32)
    for c in range(NCHUNK):
        cbt_c = cbt_ref[:, c * BK:(c + 1) * BK]
        cp = jnp.sum(cbt_c * cbt_c, axis=0, keepdims=True)
        xc2 = jnp.dot(xm2, cbt_c, preferred_element_type=jnp.float32)
        s = jnp.maximum((xp + cp) + xc2, 0.0)
        rowmin = jnp.min(s, axis=1, keepdims=True)
        iota = lax.broadcasted_iota(jnp.int32, (BN, BK), 1)
        cand = jnp.where(s == rowmin, iota, K)
        rowarg = jnp.min(cand, axis=1, keepdims=True) + c * BK
        upd = rowmin < best_val
        best_idx = jnp.where(upd, rowarg, best_idx)
        best_val = jnp.where(upd, rowmin, best_val)

    m = mask_ref[0, 0, :]
    bi = best_idx[:, 0]
    ind_ref[0, 0, :] = bi * m
    indg_ref[0, 0, :] = jnp.where(m != 0, bi, K)
    mf = m.astype(jnp.float32)

    @pl.when(i == 0)
    def _():
        acc_ref[0] = 0.0
        acc_ref[1] = 0.0

    acc_ref[0] += jnp.sum(best_val[:, 0] * mf)
    acc_ref[1] += jnp.sum(mf)

    @pl.when(i == NBLK - 1)
    def _():
        loss_ref[0, 0] = acc_ref[0] * LOSS_W / jnp.maximum(acc_ref[1], 1.0)


def _argmin_call(xm2, cbt, mask3):
    return pl.pallas_call(
        _argmin_body,
        grid=(NBLK,),
        in_specs=[
            pl.BlockSpec((BN, D), lambda i: (i, 0)),
            pl.BlockSpec((D, K), lambda i: (0, 0)),
            pl.BlockSpec((1, 1, BN), lambda i: (i, 0, 0)),
        ],
        out_specs=[
            pl.BlockSpec((1, 1, BN), lambda i: (i, 0, 0)),
            pl.BlockSpec((1, 1, BN), lambda i: (i, 0, 0)),
            pl.BlockSpec((1, 1), lambda i: (0, 0), memory_space=pltpu.SMEM),
        ],
        out_shape=[
            jax.ShapeDtypeStruct((NBLK, 1, BN), jnp.int32),
            jax.ShapeDtypeStruct((NBLK, 1, BN), jnp.int32),
            jax.ShapeDtypeStruct((1, 1), jnp.float32),
        ],
        scratch_shapes=[pltpu.SMEM((2,), jnp.float32)],
    )(xm2, cbt, mask3)


def _gather_body(table_hbm, idx_hbm, out_hbm, idx_v, rows_v, sem):
    wid = lax.axis_index("s") * SC_CORES + lax.axis_index("c")
    base = wid * BPW
    pltpu.sync_copy(idx_hbm.at[pl.ds(base, BPW)], idx_v)
    pltpu.async_copy(table_hbm.at[idx_v], rows_v, sem).wait()
    pltpu.sync_copy(rows_v, out_hbm.at[pl.ds(base, BPW)])


def _gather_call(table, idx):
    mesh = plsc.VectorSubcoreMesh(core_axis_name="c", subcore_axis_name="s")
    fn = functools.partial(
        pl.kernel,
        mesh=mesh,
        out_type=jax.ShapeDtypeStruct((N, 2 * D), jnp.float32),
        scratch_types=[
            pltpu.VMEM((BPW,), jnp.int32),
            pltpu.VMEM((BPW, 2 * D), jnp.float32),
            pltpu.SemaphoreType.DMA,
        ],
    )(_gather_body)
    return fn(table, idx)


def kernel(x, mask, codebooks):
    xm2 = (-2.0 * x).reshape(N, D)
    cbt = codebooks.T
    mask3 = mask.reshape(NBLK, 1, BN).astype(jnp.int32)
    ind3, indg3, loss11 = _argmin_call(xm2, cbt, mask3)
    table = jnp.zeros((K + 8, 2 * D), jnp.float32).at[:K, :D].set(codebooks)
    q = _gather_call(table, indg3.reshape(N))
    return (
        ind3.reshape(B, T),
        q[:, :D].reshape(B, T, D),
        loss11[0, 0],
    )
